# baseline (speedup 1.0000x reference)
import functools

import jax
import jax.numpy as jnp
from jax import lax
from jax.experimental import pallas as pl
from jax.experimental.pallas import tpu as pltpu
from jax.experimental.pallas import tpu_sc as plsc

_EMBED = 64
_PAD = 128
_SCALE = 8.0

_info = plsc.get_sparse_core_info()
_NC = _info.num_cores
_NS = _info.num_subcores
_L = _info.num_lanes
_NW = _NC * _NS

_CHUNK = 128
_NBUF = 4
_AHEAD = 2
_ROW_UNROLL = 8


def _format_table(tT, tailp, vocab):
    nblk = vocab // _CHUNK
    tail_rows = vocab - nblk * _CHUNK
    base_per_w = nblk // _NW
    extra = nblk - base_per_w * _NW

    @functools.partial(
        pl.kernel,
        mesh=plsc.VectorSubcoreMesh(core_axis_name="c", subcore_axis_name="s"),
        compiler_params=pltpu.CompilerParams(
            use_tc_tiling_on_sc=True, needs_layout_passes=False
        ),
        out_type=jax.ShapeDtypeStruct((vocab, _PAD), jnp.float32),
        scratch_types=[
            pltpu.VMEM((2, _EMBED, _CHUNK), jnp.float32),
            pltpu.VMEM((2, _CHUNK, _PAD), jnp.float32),
            pltpu.VMEM((tail_rows, _PAD), jnp.float32),
            pltpu.SemaphoreType.DMA((2,)),
            pltpu.SemaphoreType.DMA((2,)),
            pltpu.SemaphoreType.DMA,
        ],
    )
    def _k0(tT_hbm, tail_hbm, out_hbm, in_v, tr_v, tail_v, isem, osem, tsem):
        wid = lax.axis_index("s") * _NC + lax.axis_index("c")
        nb = jnp.where(wid < extra, base_per_w + 1, base_per_w)
        start = base_per_w * wid + jnp.minimum(wid, extra)
        ci = lax.iota(jnp.int32, _L)
        cvs = [ci + c0 for c0 in range(0, _EMBED, _L)]

        def start_in(i, b):
            pltpu.async_copy(
                tT_hbm.at[:, pl.ds((start + i) * _CHUNK, _CHUNK)],
                in_v.at[b], isem.at[b],
            )

        for i in range(2):
            start_in(i, i)

        def blk_body(i, carry):
            b = lax.rem(i, 2)
            pltpu.make_async_copy(
                tT_hbm.at[:, pl.ds((start + i) * _CHUNK, _CHUNK)],
                in_v.at[b], isem.at[b],
            ).wait()

            @pl.when(i >= 2)
            def _():
                pltpu.make_async_copy(
                    tr_v.at[b], out_hbm.at[pl.ds(0, _CHUNK)], osem.at[b]
                ).wait()

            def row_body(r8, carry2):
                for k in range(8):
                    r = r8 * 8 + k
                    rv = jnp.full((_L,), r, jnp.int32)
                    for j, cv in enumerate(cvs):
                        v = plsc.load_gather(in_v.at[b], [cv, rv])
                        tr_v[b, r, pl.ds(j * _L, _L)] = v * _SCALE
                return carry2

            lax.fori_loop(0, _CHUNK // 8, row_body, 0)

            pltpu.async_copy(
                tr_v.at[b],
                out_hbm.at[pl.ds((start + i) * _CHUNK, _CHUNK)],
                osem.at[b],
            )

            @pl.when(i + 2 < nb)
            def _():
                start_in(i + 2, b)

            return carry

        lax.fori_loop(0, nb, blk_body, 0)

        @pl.when(wid == 0)
        def _():
            pltpu.sync_copy(tail_hbm, tail_v)

            def trow_body(r, carry2):
                for c0 in range(0, _EMBED, _L):
                    tail_v[r, pl.ds(c0, _L)] = tail_v[r, pl.ds(c0, _L)] * _SCALE
                return carry2

            lax.fori_loop(0, tail_rows, trow_body, 0)
            pltpu.async_copy(
                tail_v, out_hbm.at[pl.ds(nblk * _CHUNK, tail_rows)], tsem
            ).wait()

        def drain_body(k, carry):
            pltpu.make_async_copy(
                tr_v.at[lax.rem(nb - 2 + k, 2)],
                out_hbm.at[pl.ds(0, _CHUNK)],
                osem.at[lax.rem(nb - 2 + k, 2)],
            ).wait()
            return carry

        lax.fori_loop(0, 2, drain_body, 0)

    return _k0(tT, tailp)


def _gather(tok2d, tscaled, B, n_chunks):
    @functools.partial(
        pl.kernel,
        mesh=plsc.VectorSubcoreMesh(core_axis_name="c", subcore_axis_name="s"),
        compiler_params=pltpu.CompilerParams(use_tc_tiling_on_sc=False),
        out_type=jax.ShapeDtypeStruct((B, _PAD), jnp.float32),
        scratch_types=[
            pltpu.VMEM((n_chunks, _CHUNK), jnp.int32),
            pltpu.VMEM((_NBUF, _CHUNK, _PAD), jnp.float32),
            pltpu.SemaphoreType.DMA((_NBUF,)),
            pltpu.SemaphoreType.DMA((_NBUF,)),
        ],
    )
    def _k2(tok_hbm, table_hbm, out_hbm, idx_v, rows_v, gsem, osem):
        wid = lax.axis_index("s") * _NC + lax.axis_index("c")
        cbase = wid * n_chunks

        pltpu.sync_copy(tok_hbm.at[pl.ds(cbase, n_chunks)], idx_v)

        def start_gather(c, b):
            pltpu.async_copy(
                table_hbm.at[idx_v.at[c]], rows_v.at[b], gsem.at[b]
            )

        for c in range(_AHEAD):
            start_gather(c, c % _NBUF)

        def chunk_body(c, carry):
            b = lax.rem(c, _NBUF)
            ca = c + _AHEAD
            ba = lax.rem(ca, _NBUF)

            @pl.when(c >= _NBUF - _AHEAD)
            def _():
                pltpu.make_async_copy(
                    rows_v.at[ba], out_hbm.at[pl.ds(0, _CHUNK)], osem.at[ba]
                ).wait()

            @pl.when(ca < n_chunks)
            def _():
                start_gather(ca, ba)

            pltpu.make_async_copy(
                table_hbm.at[idx_v.at[c]], rows_v.at[b], gsem.at[b]
            ).wait()
            pltpu.async_copy(
                rows_v.at[b], out_hbm.at[pl.ds((cbase + c) * _CHUNK, _CHUNK)],
                osem.at[b],
            )
            return carry

        lax.fori_loop(0, n_chunks, chunk_body, 0)

        for c in range(n_chunks - (_NBUF - _AHEAD), n_chunks):
            b = c % _NBUF
            pltpu.make_async_copy(
                rows_v.at[b], out_hbm.at[pl.ds(0, _CHUNK)], osem.at[b]
            ).wait()

    return _k2(tok2d, tscaled)


def kernel(tokens, table):
    B = tokens.shape[0] * tokens.shape[1]
    vocab = table.shape[0]
    n_chunks_total = B // _CHUNK
    n_chunks = n_chunks_total // _NW
    tok2d = tokens.reshape((n_chunks_total, _CHUNK)).astype(jnp.int32)

    nblk = vocab // _CHUNK
    tailp = jnp.pad(table[nblk * _CHUNK:], ((0, 0), (0, _PAD - _EMBED)))
    tscaled = _format_table(table.T, tailp, vocab)
    out = _gather(tok2d, tscaled, B, n_chunks)
    return out[:, :_EMBED].reshape(tokens.shape + (_EMBED,))


# speedup vs baseline: 1.1431x; 1.1431x over previous
import functools

import jax
import jax.numpy as jnp
from jax import lax
from jax.experimental import pallas as pl
from jax.experimental.pallas import tpu as pltpu
from jax.experimental.pallas import tpu_sc as plsc

_EMBED = 64
_PAD = 128
_SCALE = 8.0

_info = plsc.get_sparse_core_info()
_NC = _info.num_cores
_NS = _info.num_subcores
_L = _info.num_lanes
_NW = _NC * _NS

_CHUNK = 128
_NBUF = 4
_AHEAD = 2
_ROW_UNROLL = 8


def _format_table(tT, tailp, vocab):
    nblk = vocab // _CHUNK
    tail_rows = vocab - nblk * _CHUNK
    base_per_w = nblk // _NW
    extra = nblk - base_per_w * _NW

    @functools.partial(
        pl.kernel,
        mesh=plsc.VectorSubcoreMesh(core_axis_name="c", subcore_axis_name="s"),
        compiler_params=pltpu.CompilerParams(
            use_tc_tiling_on_sc=True, needs_layout_passes=False
        ),
        out_type=jax.ShapeDtypeStruct((vocab, _PAD), jnp.float32),
        scratch_types=[
            pltpu.VMEM((2, _EMBED, _CHUNK), jnp.float32),
            pltpu.VMEM((2, _CHUNK, 137), jnp.float32),
            pltpu.VMEM((tail_rows, _PAD), jnp.float32),
            pltpu.SemaphoreType.DMA((2,)),
            pltpu.SemaphoreType.DMA((2,)),
            pltpu.SemaphoreType.DMA,
        ],
    )
    def _k0(tT_hbm, tail_hbm, out_hbm, in_v, tr_v, tail_v, isem, osem, tsem):
        wid = lax.axis_index("s") * _NC + lax.axis_index("c")
        nb = jnp.where(wid < extra, base_per_w + 1, base_per_w)
        start = base_per_w * wid + jnp.minimum(wid, extra)
        ci = lax.iota(jnp.int32, _L)
        rvs = [(ci + r0, r0) for r0 in range(0, _CHUNK, _L)]

        def start_in(i, b):
            pltpu.async_copy(
                tT_hbm.at[:, pl.ds((start + i) * _CHUNK, _CHUNK)],
                in_v.at[b], isem.at[b],
            )

        for i in range(2):
            start_in(i, i)

        def blk_body(i, carry):
            b = lax.rem(i, 2)
            pltpu.make_async_copy(
                tT_hbm.at[:, pl.ds((start + i) * _CHUNK, _CHUNK)],
                in_v.at[b], isem.at[b],
            ).wait()

            @pl.when(i >= 2)
            def _():
                pltpu.make_async_copy(
                    tr_v.at[b, :, pl.ds(0, _PAD)],
                    out_hbm.at[pl.ds(0, _CHUNK)], osem.at[b]
                ).wait()

            def col_body(c, carry2):
                cv = jnp.full((_L,), c, jnp.int32)
                for rv, r0 in rvs:
                    v = in_v[b, c, pl.ds(r0, _L)]
                    plsc.store_scatter(tr_v.at[b], [rv, cv], v * _SCALE)
                return carry2

            lax.fori_loop(0, _EMBED, col_body, 0)

            pltpu.async_copy(
                tr_v.at[b, :, pl.ds(0, _PAD)],
                out_hbm.at[pl.ds((start + i) * _CHUNK, _CHUNK)],
                osem.at[b],
            )

            @pl.when(i + 2 < nb)
            def _():
                start_in(i + 2, b)

            return carry

        lax.fori_loop(0, nb, blk_body, 0)

        @pl.when(wid == 0)
        def _():
            pltpu.sync_copy(tail_hbm, tail_v)

            def trow_body(r, carry2):
                for c0 in range(0, _EMBED, _L):
                    tail_v[r, pl.ds(c0, _L)] = tail_v[r, pl.ds(c0, _L)] * _SCALE
                return carry2

            lax.fori_loop(0, tail_rows, trow_body, 0)
            pltpu.async_copy(
                tail_v, out_hbm.at[pl.ds(nblk * _CHUNK, tail_rows)], tsem
            ).wait()

        def drain_body(k, carry):
            pltpu.make_async_copy(
                tr_v.at[lax.rem(nb - 2 + k, 2), :, pl.ds(0, _PAD)],
                out_hbm.at[pl.ds(0, _CHUNK)],
                osem.at[lax.rem(nb - 2 + k, 2)],
            ).wait()
            return carry

        lax.fori_loop(0, 2, drain_body, 0)

    return _k0(tT, tailp)


def _gather(tok2d, tscaled, B, n_chunks):
    @functools.partial(
        pl.kernel,
        mesh=plsc.VectorSubcoreMesh(core_axis_name="c", subcore_axis_name="s"),
        compiler_params=pltpu.CompilerParams(use_tc_tiling_on_sc=False),
        out_type=jax.ShapeDtypeStruct((B, _PAD), jnp.float32),
        scratch_types=[
            pltpu.VMEM((n_chunks, _CHUNK), jnp.int32),
            pltpu.VMEM((_NBUF, _CHUNK, _PAD), jnp.float32),
            pltpu.SemaphoreType.DMA((_NBUF,)),
            pltpu.SemaphoreType.DMA((_NBUF,)),
        ],
    )
    def _k2(tok_hbm, table_hbm, out_hbm, idx_v, rows_v, gsem, osem):
        wid = lax.axis_index("s") * _NC + lax.axis_index("c")
        cbase = wid * n_chunks

        pltpu.sync_copy(tok_hbm.at[pl.ds(cbase, n_chunks)], idx_v)

        def start_gather(c, b):
            pltpu.async_copy(
                table_hbm.at[idx_v.at[c]], rows_v.at[b], gsem.at[b]
            )

        for c in range(_AHEAD):
            start_gather(c, c % _NBUF)

        def chunk_body(c, carry):
            b = lax.rem(c, _NBUF)
            ca = c + _AHEAD
            ba = lax.rem(ca, _NBUF)

            @pl.when(c >= _NBUF - _AHEAD)
            def _():
                pltpu.make_async_copy(
                    rows_v.at[ba], out_hbm.at[pl.ds(0, _CHUNK)], osem.at[ba]
                ).wait()

            @pl.when(ca < n_chunks)
            def _():
                start_gather(ca, ba)

            pltpu.make_async_copy(
                table_hbm.at[idx_v.at[c]], rows_v.at[b], gsem.at[b]
            ).wait()
            pltpu.async_copy(
                rows_v.at[b], out_hbm.at[pl.ds((cbase + c) * _CHUNK, _CHUNK)],
                osem.at[b],
            )
            return carry

        lax.fori_loop(0, n_chunks, chunk_body, 0)

        for c in range(n_chunks - (_NBUF - _AHEAD), n_chunks):
            b = c % _NBUF
            pltpu.make_async_copy(
                rows_v.at[b], out_hbm.at[pl.ds(0, _CHUNK)], osem.at[b]
            ).wait()

    return _k2(tok2d, tscaled)


def kernel(tokens, table):
    B = tokens.shape[0] * tokens.shape[1]
    vocab = table.shape[0]
    n_chunks_total = B // _CHUNK
    n_chunks = n_chunks_total // _NW
    tok2d = tokens.reshape((n_chunks_total, _CHUNK)).astype(jnp.int32)

    nblk = vocab // _CHUNK
    tailp = jnp.pad(table[nblk * _CHUNK:], ((0, 0), (0, _PAD - _EMBED)))
    tscaled = _format_table(table.T, tailp, vocab)
    out = _gather(tok2d, tscaled, B, n_chunks)
    return out[:, :_EMBED].reshape(tokens.shape + (_EMBED,))


# speedup vs baseline: 2.0844x; 1.8234x over previous
import functools

import jax
import jax.numpy as jnp
from jax import lax
from jax.experimental import pallas as pl
from jax.experimental.pallas import tpu as pltpu
from jax.experimental.pallas import tpu_sc as plsc

_EMBED = 64
_PAD = 128
_SCALE = 8.0

_info = plsc.get_sparse_core_info()
_NC = _info.num_cores
_NS = _info.num_subcores
_L = _info.num_lanes
_NW = _NC * _NS

_CHUNK = 128
_VECS_PER_ROW = _EMBED // _L
_NBUF = 6
_AHEAD = 3
_ROW_UNROLL = 8


def kernel(tokens, table):
    B = tokens.shape[0] * tokens.shape[1]
    n_chunks_total = B // _CHUNK
    n_chunks = n_chunks_total // _NW
    tok2d = tokens.reshape((n_chunks_total, _CHUNK)).astype(jnp.int32)
    tpad = jnp.pad(table, ((0, 0), (0, _PAD - _EMBED)))

    @functools.partial(
        pl.kernel,
        mesh=plsc.VectorSubcoreMesh(core_axis_name="c", subcore_axis_name="s"),
        compiler_params=pltpu.CompilerParams(use_tc_tiling_on_sc=False),
        out_type=jax.ShapeDtypeStruct((B, _PAD), jnp.float32),
        scratch_types=[
            pltpu.VMEM((n_chunks, _CHUNK), jnp.int32),
            pltpu.VMEM((_NBUF, _CHUNK, _PAD), jnp.float32),
            pltpu.SemaphoreType.DMA((_NBUF,)),
            pltpu.SemaphoreType.DMA((_NBUF,)),
        ],
    )
    def _emb(tok_hbm, table_hbm, out_hbm, idx_v, rows_v, gsem, osem):
        wid = lax.axis_index("s") * _NC + lax.axis_index("c")
        cbase = wid * n_chunks

        pltpu.sync_copy(tok_hbm.at[pl.ds(cbase, n_chunks)], idx_v)

        def start_gather(c, b):
            pltpu.async_copy(
                table_hbm.at[idx_v.at[c]], rows_v.at[b], gsem.at[b]
            )

        for c in range(_AHEAD):
            start_gather(c, c % _NBUF)

        def chunk_body(c, carry):
            b = lax.rem(c, _NBUF)
            ca = c + _AHEAD
            ba = lax.rem(ca, _NBUF)

            @pl.when(c >= _NBUF - _AHEAD)
            def _():
                pltpu.make_async_copy(
                    rows_v.at[ba], out_hbm.at[pl.ds(0, _CHUNK)], osem.at[ba]
                ).wait()

            @pl.when(ca < n_chunks)
            def _():
                start_gather(ca, ba)

            pltpu.make_async_copy(
                table_hbm.at[idx_v.at[c]], rows_v.at[b], gsem.at[b]
            ).wait()

            def row_body(i, carry2):
                for k in range(_ROW_UNROLL):
                    r = i * _ROW_UNROLL + k
                    row = r // _VECS_PER_ROW
                    v = r % _VECS_PER_ROW
                    rows_v[b, row, pl.ds(v * _L, _L)] = (
                        rows_v[b, row, pl.ds(v * _L, _L)] * _SCALE
                    )
                return carry2

            lax.fori_loop(0, _CHUNK * _VECS_PER_ROW // _ROW_UNROLL,
                          row_body, 0)

            pltpu.async_copy(
                rows_v.at[b], out_hbm.at[pl.ds((cbase + c) * _CHUNK, _CHUNK)],
                osem.at[b],
            )
            return carry

        lax.fori_loop(0, n_chunks, chunk_body, 0)

        for c in range(n_chunks - (_NBUF - _AHEAD), n_chunks):
            b = c % _NBUF
            pltpu.make_async_copy(
                rows_v.at[b], out_hbm.at[pl.ds(0, _CHUNK)], osem.at[b]
            ).wait()

    out = _emb(tok2d, tpad)
    return out[:, :_EMBED].reshape(tokens.shape + (_EMBED,))


# speedup vs baseline: 2.0935x; 1.0044x over previous
import functools

import jax
import jax.numpy as jnp
from jax import lax
from jax.experimental import pallas as pl
from jax.experimental.pallas import tpu as pltpu
from jax.experimental.pallas import tpu_sc as plsc

_EMBED = 64
_PAD = 128
_SCALE = 8.0

_info = plsc.get_sparse_core_info()
_NC = _info.num_cores
_NS = _info.num_subcores
_L = _info.num_lanes
_NW = _NC * _NS

_CHUNK = 128
_VECS_PER_ROW = _EMBED // _L
_NBUF = 4
_AHEAD = 2
_ROW_UNROLL = 8


def kernel(tokens, table):
    B = tokens.shape[0] * tokens.shape[1]
    n_chunks_total = B // _CHUNK
    n_chunks = n_chunks_total // _NW
    tok2d = tokens.reshape((n_chunks_total, _CHUNK)).astype(jnp.int32)
    tpad = jnp.pad(table, ((0, 0), (0, _PAD - _EMBED)))

    @functools.partial(
        pl.kernel,
        mesh=plsc.VectorSubcoreMesh(core_axis_name="c", subcore_axis_name="s"),
        compiler_params=pltpu.CompilerParams(use_tc_tiling_on_sc=False),
        out_type=jax.ShapeDtypeStruct((B, _PAD), jnp.float32),
        scratch_types=[
            pltpu.VMEM((n_chunks, _CHUNK), jnp.int32),
            pltpu.VMEM((_NBUF, _CHUNK, _PAD), jnp.float32),
            pltpu.SemaphoreType.DMA((_NBUF,)),
            pltpu.SemaphoreType.DMA((_NBUF,)),
        ],
    )
    def _emb(tok_hbm, table_hbm, out_hbm, idx_v, rows_v, gsem, osem):
        wid = lax.axis_index("s") * _NC + lax.axis_index("c")
        cbase = wid * n_chunks

        pltpu.sync_copy(tok_hbm.at[pl.ds(cbase, n_chunks)], idx_v)

        def start_gather(c, b):
            pltpu.async_copy(
                table_hbm.at[idx_v.at[c]], rows_v.at[b], gsem.at[b]
            )

        for c in range(_AHEAD):
            start_gather(c, c % _NBUF)

        def chunk_body(c, carry):
            b = lax.rem(c, _NBUF)
            ca = c + _AHEAD
            ba = lax.rem(ca, _NBUF)

            @pl.when(c >= _NBUF - _AHEAD)
            def _():
                pltpu.make_async_copy(
                    rows_v.at[ba], out_hbm.at[pl.ds(0, _CHUNK)], osem.at[ba]
                ).wait()

            @pl.when(ca < n_chunks)
            def _():
                start_gather(ca, ba)

            pltpu.make_async_copy(
                table_hbm.at[idx_v.at[c]], rows_v.at[b], gsem.at[b]
            ).wait()

            def row_body(i, carry2):
                for k in range(_ROW_UNROLL):
                    r = i * _ROW_UNROLL + k
                    row = r // _VECS_PER_ROW
                    v = r % _VECS_PER_ROW
                    rows_v[b, row, pl.ds(v * _L, _L)] = (
                        rows_v[b, row, pl.ds(v * _L, _L)] * _SCALE
                    )
                return carry2

            lax.fori_loop(0, _CHUNK * _VECS_PER_ROW // _ROW_UNROLL,
                          row_body, 0)

            pltpu.async_copy(
                rows_v.at[b], out_hbm.at[pl.ds((cbase + c) * _CHUNK, _CHUNK)],
                osem.at[b],
            )
            return carry

        lax.fori_loop(0, n_chunks, chunk_body, 0)

        for c in range(n_chunks - (_NBUF - _AHEAD), n_chunks):
            b = c % _NBUF
            pltpu.make_async_copy(
                rows_v.at[b], out_hbm.at[pl.ds(0, _CHUNK)], osem.at[b]
            ).wait()

    out = _emb(tok2d, tpad)
    return out[:, :_EMBED].reshape(tokens.shape + (_EMBED,))
